# Initial kernel scaffold; baseline (speedup 1.0000x reference)
import jax
import jax.numpy as jnp
from jax.experimental import pallas as pl


def kernel(x, coords, emb1, emb2):
    raise NotImplementedError("write your pallas kernel here")


# speedup vs baseline: 3.0572x; 3.0572x over previous
import functools
import jax
import jax.numpy as jnp
from jax import lax
from jax.experimental import pallas as pl
from jax.experimental.pallas import tpu as pltpu
from jax.experimental.pallas import tpu_sc as plsc

TILE = 16
SEQ = 65536
DIM = 128
HALF = DIM // 2
NTAB = 512
NC, NS, L = 2, 16, 16
NW = NC * NS
N2 = 2 * SEQ
CHUNK = N2 // NW
VPC = CHUNK // L
BLK = 128
NBLK = CHUNK // BLK

_mesh = plsc.VectorSubcoreMesh(
    core_axis_name="c", subcore_axis_name="s", num_cores=NC, num_subcores=NS
)


def _wid():
    return lax.axis_index("s") * NC + lax.axis_index("c")


def _lane_shuffle(v, idx):
    return lax.gather(
        v,
        idx[:, None],
        dimension_numbers=lax.GatherDimensionNumbers(
            offset_dims=(), collapsed_slice_dims=(0,), start_index_map=(0,)
        ),
        slice_sizes=(1,),
        mode=lax.GatherScatterMode.PROMISE_IN_BOUNDS,
    )


@functools.partial(
    pl.kernel,
    out_type=(
        jax.ShapeDtypeStruct((NW, L), jnp.int32),
        jax.ShapeDtypeStruct((2 * NTAB, HALF), jnp.float32),
    ),
    mesh=_mesh,
    scratch_types=[
        pltpu.VMEM((CHUNK,), jnp.int32),
        pltpu.VMEM((L,), jnp.int32),
    ],
)
def _prep_kernel(coords_hbm, emb1_hbm, emb2_hbm, minmat_hbm, table_hbm,
                 cbuf, mv):
    wid = _wid()

    rows = 2 * NTAB // NW

    @pl.when(wid < NW // 2)
    def _():
        pltpu.sync_copy(emb1_hbm.at[pl.ds(wid * rows, rows)],
                        table_hbm.at[pl.ds(wid * rows, rows)])

    @pl.when(wid >= NW // 2)
    def _():
        pltpu.sync_copy(emb2_hbm.at[pl.ds(wid * rows - NTAB, rows)],
                        table_hbm.at[pl.ds(wid * rows, rows)])

    pltpu.sync_copy(coords_hbm.at[pl.ds(wid * CHUNK, CHUNK)], cbuf)

    def body(i, m):
        return jnp.minimum(m, cbuf[pl.ds(i * L, L)])

    m = lax.fori_loop(1, VPC, body, cbuf[pl.ds(0, L)])

    iota = lax.iota(jnp.int32, L)
    for d in (2, 4, 8):
        g = _lane_shuffle(m, jnp.bitwise_xor(iota, d))
        m = jnp.minimum(m, g)
    mv[...] = m
    pltpu.sync_copy(mv, minmat_hbm.at[wid])


@functools.partial(
    pl.kernel,
    out_type=jax.ShapeDtypeStruct((N2, HALF), jnp.float32),
    mesh=_mesh,
    scratch_types=[
        pltpu.VMEM((CHUNK,), jnp.int32),
        pltpu.VMEM((CHUNK,), jnp.int32),
        pltpu.VMEM((NW, L), jnp.int32),
        pltpu.VMEM((BLK, HALF), jnp.float32),
        pltpu.SemaphoreType.DMA,
    ],
    compiler_params=pltpu.CompilerParams(use_tc_tiling_on_sc=False),
)
def _emb_kernel(x_hbm, coords_hbm, table_hbm, minmat_hbm, out_hbm,
                cbuf, idxbuf, mbuf, xbuf, sem):
    wid = _wid()
    base = wid * CHUNK

    pltpu.sync_copy(coords_hbm.at[pl.ds(base, CHUNK)], cbuf)
    pltpu.sync_copy(minmat_hbm, mbuf)

    def mbody(i, m):
        return jnp.minimum(m, mbuf[i, :])

    m = lax.fori_loop(1, NW, mbody, mbuf[0, :])

    offs = jnp.bitwise_and(lax.iota(jnp.int32, L), 1) * NTAB

    @pl.loop(0, VPC)
    def _(i):
        c = cbuf[pl.ds(i * L, L)]
        idxbuf[pl.ds(i * L, L)] = (
            lax.shift_right_arithmetic(c - m, 4) + offs
        )

    @pl.loop(0, NBLK)
    def _(j):
        row0 = base + j * BLK
        pltpu.sync_copy(x_hbm.at[pl.ds(row0, BLK)], xbuf)
        pltpu.async_copy(
            table_hbm.at[idxbuf.at[pl.ds(j * BLK, BLK)]], xbuf, sem,
            add=True,
        ).wait()
        pltpu.sync_copy(xbuf, out_hbm.at[pl.ds(row0, BLK)])


def kernel(x, coords, emb1, emb2):
    coords_flat = coords.reshape(N2)
    minmat, table = _prep_kernel(coords_flat, emb1, emb2)
    x2 = x.reshape(N2, HALF)
    out2 = _emb_kernel(x2, coords_flat, table, minmat)
    return out2.reshape(SEQ, DIM)


# speedup vs baseline: 3.6216x; 1.1846x over previous
import functools
import jax
import jax.numpy as jnp
from jax import lax
from jax.experimental import pallas as pl
from jax.experimental.pallas import tpu as pltpu
from jax.experimental.pallas import tpu_sc as plsc

TILE = 16
SEQ = 65536
DIM = 128
HALF = DIM // 2
NTAB = 512
NC, NS, L = 2, 16, 16
NW = NC * NS
N2 = 2 * SEQ
CHUNK = N2 // NW
VPC = CHUNK // L
BLK = 128
NBLK = CHUNK // BLK
NBUF = 4

_mesh = plsc.VectorSubcoreMesh(
    core_axis_name="c", subcore_axis_name="s", num_cores=NC, num_subcores=NS
)


def _wid():
    return lax.axis_index("s") * NC + lax.axis_index("c")


def _lane_shuffle(v, idx):
    return lax.gather(
        v,
        idx[:, None],
        dimension_numbers=lax.GatherDimensionNumbers(
            offset_dims=(), collapsed_slice_dims=(0,), start_index_map=(0,)
        ),
        slice_sizes=(1,),
        mode=lax.GatherScatterMode.PROMISE_IN_BOUNDS,
    )


@functools.partial(
    pl.kernel,
    out_type=(
        jax.ShapeDtypeStruct((NW, L), jnp.int32),
        jax.ShapeDtypeStruct((2 * NTAB, HALF), jnp.float32),
    ),
    mesh=_mesh,
    scratch_types=[
        pltpu.VMEM((CHUNK,), jnp.int32),
        pltpu.VMEM((L,), jnp.int32),
    ],
)
def _prep_kernel(coords_hbm, emb1_hbm, emb2_hbm, minmat_hbm, table_hbm,
                 cbuf, mv):
    wid = _wid()

    rows = 2 * NTAB // NW

    @pl.when(wid < NW // 2)
    def _():
        pltpu.sync_copy(emb1_hbm.at[pl.ds(wid * rows, rows)],
                        table_hbm.at[pl.ds(wid * rows, rows)])

    @pl.when(wid >= NW // 2)
    def _():
        pltpu.sync_copy(emb2_hbm.at[pl.ds(wid * rows - NTAB, rows)],
                        table_hbm.at[pl.ds(wid * rows, rows)])

    pltpu.sync_copy(coords_hbm.at[pl.ds(wid * CHUNK, CHUNK)], cbuf)

    def body(i, m):
        return jnp.minimum(m, cbuf[pl.ds(i * L, L)])

    m = lax.fori_loop(1, VPC, body, cbuf[pl.ds(0, L)])

    iota = lax.iota(jnp.int32, L)
    for d in (2, 4, 8):
        g = _lane_shuffle(m, jnp.bitwise_xor(iota, d))
        m = jnp.minimum(m, g)
    mv[...] = m
    pltpu.sync_copy(mv, minmat_hbm.at[wid])


@functools.partial(
    pl.kernel,
    out_type=jax.ShapeDtypeStruct((N2, HALF), jnp.float32),
    mesh=_mesh,
    scratch_types=[
        pltpu.VMEM((CHUNK,), jnp.int32),
        pltpu.VMEM((CHUNK,), jnp.int32),
        pltpu.VMEM((NW, L), jnp.int32),
        pltpu.VMEM((NBUF, BLK, HALF), jnp.float32),
        pltpu.SemaphoreType.DMA((NBUF,)),
        pltpu.SemaphoreType.DMA((NBUF,)),
        pltpu.SemaphoreType.DMA((NBUF,)),
    ],
    compiler_params=pltpu.CompilerParams(use_tc_tiling_on_sc=False),
)
def _emb_kernel(x_hbm, coords_hbm, table_hbm, minmat_hbm, out_hbm,
                cbuf, idxbuf, mbuf, xbuf, lsem, gsem, ssem):
    wid = _wid()
    base = wid * CHUNK

    pltpu.sync_copy(coords_hbm.at[pl.ds(base, CHUNK)], cbuf)
    pltpu.sync_copy(minmat_hbm, mbuf)

    def mbody(i, m):
        return jnp.minimum(m, mbuf[i, :])

    m = lax.fori_loop(1, NW, mbody, mbuf[0, :])

    offs = jnp.bitwise_and(lax.iota(jnp.int32, L), 1) * NTAB

    @pl.loop(0, VPC)
    def _(i):
        c = cbuf[pl.ds(i * L, L)]
        idxbuf[pl.ds(i * L, L)] = (
            lax.shift_right_arithmetic(c - m, 4) + offs
        )

    @pl.loop(0, NBLK + 2)
    def _(j):
        @pl.when(j >= 2)
        def _():
            jj = j - 2
            b = jj & (NBUF - 1)
            pltpu.make_async_copy(
                x_hbm.at[pl.ds(base + jj * BLK, BLK)], xbuf.at[b],
                gsem.at[b],
            ).wait()
            pltpu.async_copy(
                xbuf.at[b], out_hbm.at[pl.ds(base + jj * BLK, BLK)],
                ssem.at[b],
            )

        @pl.when((j >= 1) & (j <= NBLK))
        def _():
            jj = j - 1
            b = jj & (NBUF - 1)
            pltpu.make_async_copy(
                x_hbm.at[pl.ds(base + jj * BLK, BLK)], xbuf.at[b],
                lsem.at[b],
            ).wait()
            pltpu.async_copy(
                table_hbm.at[idxbuf.at[pl.ds(jj * BLK, BLK)]], xbuf.at[b],
                gsem.at[b], add=True,
            )

        @pl.when(j < NBLK)
        def _():
            b = j & (NBUF - 1)

            @pl.when(j >= NBUF)
            def _():
                pltpu.make_async_copy(
                    xbuf.at[b],
                    out_hbm.at[pl.ds(base + (j - NBUF) * BLK, BLK)],
                    ssem.at[b],
                ).wait()

            pltpu.async_copy(
                x_hbm.at[pl.ds(base + j * BLK, BLK)], xbuf.at[b],
                lsem.at[b],
            )

    @pl.loop(NBLK, NBLK + NBUF)
    def _(j):
        b = j & (NBUF - 1)
        pltpu.make_async_copy(
            xbuf.at[b], out_hbm.at[pl.ds(base + (j - NBUF) * BLK, BLK)],
            ssem.at[b],
        ).wait()


def kernel(x, coords, emb1, emb2):
    coords_flat = coords.reshape(N2)
    minmat, table = _prep_kernel(coords_flat, emb1, emb2)
    x2 = x.reshape(N2, HALF)
    out2 = _emb_kernel(x2, coords_flat, table, minmat)
    return out2.reshape(SEQ, DIM)


# speedup vs baseline: 4.8966x; 1.3521x over previous
import functools
import jax
import jax.numpy as jnp
from jax import lax
from jax.experimental import pallas as pl
from jax.experimental.pallas import tpu as pltpu
from jax.experimental.pallas import tpu_sc as plsc

TILE = 16
SEQ = 65536
DIM = 128
HALF = DIM // 2
NTAB = 512
NC, NS, L = 2, 16, 16
NW = NC * NS
N2 = 2 * SEQ
CHUNK = N2 // NW
VPC = CHUNK // L
SCAN = N2 // NS
VPS = SCAN // L
BLK = 128
NBLK = CHUNK // BLK
NBUF = 4
TROWS = NTAB // NS

_mesh = plsc.VectorSubcoreMesh(
    core_axis_name="c", subcore_axis_name="s", num_cores=NC, num_subcores=NS
)


def _lane_shuffle(v, idx):
    return lax.gather(
        v,
        idx[:, None],
        dimension_numbers=lax.GatherDimensionNumbers(
            offset_dims=(), collapsed_slice_dims=(0,), start_index_map=(0,)
        ),
        slice_sizes=(1,),
        mode=lax.GatherScatterMode.PROMISE_IN_BOUNDS,
    )


@functools.partial(
    pl.kernel,
    out_type=jax.ShapeDtypeStruct((N2, HALF), jnp.float32),
    mesh=_mesh,
    scratch_types=[
        pltpu.VMEM((SCAN,), jnp.int32),
        pltpu.VMEM((CHUNK,), jnp.int32),
        pltpu.VMEM((NS, L), jnp.int32),
        pltpu.VMEM((L,), jnp.int32),
        pltpu.VMEM((NBUF, BLK, HALF), jnp.float32),
        pltpu.VMEM_SHARED((2 * NTAB, HALF), jnp.float32),
        pltpu.VMEM_SHARED((NS, L), jnp.int32),
        pltpu.SemaphoreType.DMA((NBUF,)),
        pltpu.SemaphoreType.DMA((NBUF,)),
        pltpu.SemaphoreType.DMA((NBUF,)),
    ],
    compiler_params=pltpu.CompilerParams(use_tc_tiling_on_sc=False),
)
def _emb_kernel(x_hbm, coords_hbm, emb1_hbm, emb2_hbm, out_hbm,
                cbuf, idxbuf, mbuf, mv, xbuf, tab_sh, min_sh,
                lsem, gsem, ssem):
    cid = lax.axis_index("c")
    sid = lax.axis_index("s")
    wid = sid * NC + cid
    base = wid * CHUNK

    pltpu.sync_copy(emb1_hbm.at[pl.ds(sid * TROWS, TROWS)],
                    tab_sh.at[pl.ds(sid * TROWS, TROWS)])
    pltpu.sync_copy(emb2_hbm.at[pl.ds(sid * TROWS, TROWS)],
                    tab_sh.at[pl.ds(NTAB + sid * TROWS, TROWS)])

    pltpu.sync_copy(coords_hbm.at[pl.ds(sid * SCAN, SCAN)], cbuf)

    def body(i, m):
        return jnp.minimum(m, cbuf[pl.ds(i * L, L)])

    m = lax.fori_loop(1, VPS, body, cbuf[pl.ds(0, L)])
    mv[...] = m
    pltpu.sync_copy(mv, min_sh.at[sid])
    plsc.subcore_barrier()

    pltpu.sync_copy(min_sh, mbuf)

    def mbody(i, m):
        return jnp.minimum(m, mbuf[i, :])

    m = lax.fori_loop(1, NS, mbody, mbuf[0, :])
    iota = lax.iota(jnp.int32, L)
    for d in (2, 4, 8):
        m = jnp.minimum(m, _lane_shuffle(m, jnp.bitwise_xor(iota, d)))

    offs = jnp.bitwise_and(iota, 1) * NTAB
    cb = cid * CHUNK

    @pl.loop(0, VPC)
    def _(i):
        c = cbuf[pl.ds(cb + i * L, L)]
        idxbuf[pl.ds(i * L, L)] = (
            lax.shift_right_arithmetic(c - m, 4) + offs
        )

    @pl.loop(0, NBLK + 2)
    def _(j):
        @pl.when(j >= 2)
        def _():
            jj = j - 2
            b = jj & (NBUF - 1)
            pltpu.make_async_copy(
                x_hbm.at[pl.ds(base + jj * BLK, BLK)], xbuf.at[b],
                gsem.at[b],
            ).wait()
            pltpu.async_copy(
                xbuf.at[b], out_hbm.at[pl.ds(base + jj * BLK, BLK)],
                ssem.at[b],
            )

        @pl.when((j >= 1) & (j <= NBLK))
        def _():
            jj = j - 1
            b = jj & (NBUF - 1)
            pltpu.make_async_copy(
                x_hbm.at[pl.ds(base + jj * BLK, BLK)], xbuf.at[b],
                lsem.at[b],
            ).wait()
            pltpu.async_copy(
                tab_sh.at[idxbuf.at[pl.ds(jj * BLK, BLK)]], xbuf.at[b],
                gsem.at[b], add=True,
            )

        @pl.when(j < NBLK)
        def _():
            b = j & (NBUF - 1)

            @pl.when(j >= NBUF)
            def _():
                pltpu.make_async_copy(
                    xbuf.at[b],
                    out_hbm.at[pl.ds(base + (j - NBUF) * BLK, BLK)],
                    ssem.at[b],
                ).wait()

            pltpu.async_copy(
                x_hbm.at[pl.ds(base + j * BLK, BLK)], xbuf.at[b],
                lsem.at[b],
            )

    @pl.loop(NBLK, NBLK + NBUF)
    def _(j):
        b = j & (NBUF - 1)
        pltpu.make_async_copy(
            xbuf.at[b], out_hbm.at[pl.ds(base + (j - NBUF) * BLK, BLK)],
            ssem.at[b],
        ).wait()


def kernel(x, coords, emb1, emb2):
    coords_flat = coords.reshape(N2)
    x2 = x.reshape(N2, HALF)
    out2 = _emb_kernel(x2, coords_flat, emb1, emb2)
    return out2.reshape(SEQ, DIM)
